# baseline (speedup 1.0000x reference)
import jax
import jax.numpy as jnp
import numpy as np
from jax import lax
from jax.experimental import pallas as pl
from jax.experimental.pallas import tpu as pltpu
from jax.experimental.pallas import tpu_sc as plsc

N = 10000
E = 160000
A = 128
F = 64
NB = 8
RMAX = 5.0
CORR = 3
SH = 9
AVG = 16.0

CHUNK = 1280
NCHUNK = E // CHUNK
NPAIR = CHUNK // 32
assert CHUNK % 256 == 0 and E % CHUNK == 0
GOFK = [0, 1, 1, 1, 2, 2, 2, 2, 2]


def _sph(u):
    x, y, z = u[..., 0], u[..., 1], u[..., 2]
    c1 = np.sqrt(3.0)
    c2 = np.sqrt(15.0)
    c3 = np.sqrt(5.0) / 2.0
    return jnp.stack([
        jnp.ones_like(x),
        c1 * x, c1 * y, c1 * z,
        c2 * x * y, c2 * y * z, c3 * (3.0 * z ** 2 - 1.0), c2 * x * z, (c2 / 2.0) * (x ** 2 - y ** 2),
    ], axis=-1)


def _bessel(r):
    n = jnp.arange(1, NB + 1, dtype=jnp.float32)
    r_ = r[..., None]
    b = jnp.sqrt(2.0 / RMAX) * jnp.sin(n * jnp.pi * r_ / RMAX) / (r_ + 1e-9)
    cut = 0.5 * (jnp.cos(jnp.pi * jnp.clip(r_ / RMAX, 0.0, 1.0)) + 1.0)
    return b * cut


def _interleave_rows(x):
    return x.reshape(-1, 2, 16).swapaxes(1, 2).reshape(x.shape)


def _sc_agg_body(m_hbm, r_hbm, y_hbm, idx_hbm, out_hbm,
                 m_v, r_v, y_v, idx_v, agg_v, sem0, sem1):
    cid = lax.axis_index("c")
    sid = lax.axis_index("s")
    wid = sid * 2 + cid
    sems = (sem0, sem1)

    for pass_i in range(2):
        f = pass_i * 32 + wid

        zero16 = jnp.zeros((16,), jnp.float32)

        @plsc.parallel_loop(0, (SH * N) // 16, 1, unroll=8)
        def zero_body(i):
            agg_v[pl.ds(i * 16, 16)] = zero16

        pltpu.sync_copy(m_hbm.at[pl.ds(f * N, N)], m_v)

        def chunk_start(c, b):
            pltpu.async_copy(
                r_hbm.at[pl.ds(c * (3 * F * CHUNK) + f * (3 * CHUNK), 3 * CHUNK)],
                r_v.at[pl.ds(b * 3 * CHUNK, 3 * CHUNK)], sems[b])
            pltpu.async_copy(y_hbm.at[pl.ds(c * SH * CHUNK, SH * CHUNK)],
                             y_v.at[pl.ds(b * SH * CHUNK, SH * CHUNK)], sems[b])
            pltpu.async_copy(idx_hbm.at[pl.ds(c * CHUNK, CHUNK)],
                             idx_v.at[pl.ds(b * CHUNK, CHUNK)], sems[b])

        def chunk_wait(b):
            pltpu.make_async_copy(r_hbm.at[pl.ds(0, 3 * CHUNK)], r_v.at[pl.ds(b * 3 * CHUNK, 3 * CHUNK)], sems[b]).wait()
            pltpu.make_async_copy(y_hbm.at[pl.ds(0, SH * CHUNK)], y_v.at[pl.ds(b * SH * CHUNK, SH * CHUNK)], sems[b]).wait()
            pltpu.make_async_copy(idx_hbm.at[pl.ds(0, CHUNK)], idx_v.at[pl.ds(b * CHUNK, CHUNK)], sems[b]).wait()

        def process_chunk(b):
            @plsc.parallel_loop(0, NPAIR, 1, unroll=2)
            def grp_body(j):
                pk0 = idx_v[pl.ds(b * CHUNK + j * 32, 16)]
                pk1 = idx_v[pl.ds(b * CHUNK + j * 32 + 16, 16)]
                snd0 = pk0 & 0xFFFF
                rcv0 = lax.shift_right_logical(pk0, 16)
                snd1 = pk1 & 0xFFFF
                rcv1 = lax.shift_right_logical(pk1, 16)
                mf0 = plsc.load_gather(m_v, [snd0])
                mf1 = plsc.load_gather(m_v, [snd1])
                p0 = []
                p1 = []
                for g in range(3):
                    r32 = r_v[pl.ds((b * 3 + g) * CHUNK + j * 32, 32)]
                    ra, rb = plsc.unpack(r32, format=plsc.PackFormat.INTERLEAVED)
                    p0.append(mf0 * ra)
                    p1.append(mf1 * rb)
                for k in range(SH):
                    y32 = y_v[pl.ds((b * SH + k) * CHUNK + j * 32, 32)]
                    ya, yb = plsc.unpack(y32, format=plsc.PackFormat.INTERLEAVED)
                    slab = agg_v.at[pl.ds(k * N, N)]
                    plsc.addupdate_scatter(slab, [rcv0], p0[GOFK[k]] * ya)
                    plsc.addupdate_scatter(slab, [rcv1], p1[GOFK[k]] * yb)

        chunk_start(0, 0)
        chunk_start(1, 1)

        def pair_body(i, _):
            for b in range(2):
                c = 2 * i + b
                chunk_wait(b)
                process_chunk(b)

                @pl.when(c + 2 < NCHUNK)
                def _():
                    chunk_start(c + 2, b)
            return 0

        lax.fori_loop(0, NCHUNK // 2, pair_body, 0)
        if NCHUNK % 2:
            chunk_wait(0)
            process_chunk(0)

        pltpu.sync_copy(agg_v, out_hbm.at[pl.ds(f * SH * N, SH * N)])


_sc_agg = pl.kernel(
    _sc_agg_body,
    out_type=jax.ShapeDtypeStruct((F * SH * N,), jnp.float32),
    mesh=plsc.VectorSubcoreMesh(core_axis_name="c", subcore_axis_name="s"),
    compiler_params=pltpu.CompilerParams(needs_layout_passes=False),
    scratch_types=[
        pltpu.VMEM((N,), jnp.float32),
        pltpu.VMEM((2 * 3 * CHUNK,), jnp.bfloat16),
        pltpu.VMEM((2 * SH * CHUNK,), jnp.bfloat16),
        pltpu.VMEM((2 * CHUNK,), jnp.int32),
        pltpu.VMEM((SH * N,), jnp.float32),
        pltpu.SemaphoreType.DMA,
        pltpu.SemaphoreType.DMA,
    ],
)


def _head_kernel(nf1_0_ref, nf2_0_ref, Wro_ref, Wm1_ref, Wm2_ref, out_ref):
    out0 = nf1_0_ref[...] @ Wro_ref[...]
    h = nf2_0_ref[...] @ Wm1_ref[...]
    h = h * jax.nn.sigmoid(h)
    out1 = h @ Wm2_ref[...]
    out_ref[...] = jnp.stack([out0, out1], axis=1)


def kernel(positions, node_attrs, shifts, senders, receivers, W_embed, Wr1_0, Wr2_0, Wup_0, Wsc_0, Wprod_0, Wpattr_0, Wro_0, Wr1_1, Wr2_1, Wup_1, Wsc_1, Wprod_1, Wpattr_1, Wm1, Wm2):
    vec = positions[receivers] - positions[senders] + shifts
    r = jnp.sqrt(jnp.sum(vec ** 2, axis=-1) + 1e-18)
    u = vec / r[:, None]
    Y = _sph(u)
    ef = _bessel(r)
    h0 = node_attrs @ W_embed

    y_blocked = _interleave_rows(
        Y.T.astype(jnp.bfloat16).reshape(SH, NCHUNK, CHUNK).transpose(1, 0, 2)
    ).reshape(-1)
    packed = jnp.bitwise_or(senders, receivers << 16)
    idx_blocked = packed.reshape(NCHUNK, CHUNK).reshape(-1)

    def interaction(h_scal, Wr1, Wr2, Wup, Wsc, Wprod, Wpattr):
        m_t = (h_scal @ Wup).T.reshape(-1)
        S = jax.nn.silu(ef @ Wr1)
        R_t = (S @ Wr2).T
        R_blk = _interleave_rows(
            R_t.astype(jnp.bfloat16).reshape(3 * F, NCHUNK, CHUNK).transpose(1, 0, 2)
        ).reshape(-1)
        agg = _sc_agg(m_t, R_blk, y_blocked, idx_blocked)
        agg = agg.reshape(F, SH, N).transpose(2, 0, 1) / AVG
        sc = jnp.einsum('na,afk->nfk', node_attrs, Wsc)
        inv = jnp.mean(agg ** 2, axis=-1)
        wz = node_attrs @ Wpattr
        acc = jnp.zeros_like(agg)
        for nu in range(CORR):
            acc = acc + Wprod[nu][None, :, None] * agg * (inv[:, :, None] ** nu)
        return wz[:, :, None] * acc + sc

    nf1 = interaction(h0, Wr1_0, Wr2_0, Wup_0, Wsc_0, Wprod_0, Wpattr_0)
    nf2 = interaction(nf1[:, :, 0], Wr1_1, Wr2_1, Wup_1, Wsc_1, Wprod_1, Wpattr_1)

    out = pl.pallas_call(
        _head_kernel,
        out_shape=jax.ShapeDtypeStruct((N, 2, 1), jnp.float32),
    )(nf1[:, :, 0], nf2[:, :, 0], Wro_0, Wm1, Wm2)
    return out


# speedup vs baseline: 7.6960x; 7.6960x over previous
import jax
import jax.numpy as jnp
import numpy as np
from jax import lax
from jax.experimental import pallas as pl
from jax.experimental.pallas import tpu as pltpu
from jax.experimental.pallas import tpu_sc as plsc

N = 10000
E = 160000
A = 128
F = 64
NB = 8
RMAX = 5.0
CORR = 3
SH = 9
AVG = 16.0

CHUNK = 800
NCHUNK = E // CHUNK
NGRP = CHUNK // 16
assert CHUNK % 16 == 0 and E % CHUNK == 0 and NCHUNK % 2 == 0
GOFK = [0, 1, 1, 1, 2, 2, 2, 2, 2]


def _sph(u):
    x, y, z = u[..., 0], u[..., 1], u[..., 2]
    c1 = np.sqrt(3.0)
    c2 = np.sqrt(15.0)
    c3 = np.sqrt(5.0) / 2.0
    return jnp.stack([
        jnp.ones_like(x),
        c1 * x, c1 * y, c1 * z,
        c2 * x * y, c2 * y * z, c3 * (3.0 * z ** 2 - 1.0), c2 * x * z, (c2 / 2.0) * (x ** 2 - y ** 2),
    ], axis=-1)


def _bessel(r):
    n = jnp.arange(1, NB + 1, dtype=jnp.float32)
    r_ = r[..., None]
    b = jnp.sqrt(2.0 / RMAX) * jnp.sin(n * jnp.pi * r_ / RMAX) / (r_ + 1e-9)
    cut = 0.5 * (jnp.cos(jnp.pi * jnp.clip(r_ / RMAX, 0.0, 1.0)) + 1.0)
    return b * cut


def _sc_agg_body(m_hbm, r_hbm, y_hbm, idx_hbm, out_hbm,
                 m_v, r_v, y_v, idx_v, agg_v, sem0, sem1):
    cid = lax.axis_index("c")
    sid = lax.axis_index("s")
    wid = sid * 2 + cid
    sems = (sem0, sem1)

    for pass_i in range(2):
        f = pass_i * 32 + wid

        zero16 = jnp.zeros((16,), jnp.float32)

        @plsc.parallel_loop(0, (SH * N) // 16, 1, unroll=8)
        def zero_body(i):
            agg_v[pl.ds(i * 16, 16)] = zero16

        pltpu.sync_copy(m_hbm.at[pl.ds(f * N, N)], m_v)

        def chunk_start(c, b):
            pltpu.async_copy(
                r_hbm.at[pl.ds(c * (3 * F * CHUNK) + f * (3 * CHUNK), 3 * CHUNK)],
                r_v.at[pl.ds(b * 3 * CHUNK, 3 * CHUNK)], sems[b])
            pltpu.async_copy(y_hbm.at[pl.ds(c * SH * CHUNK, SH * CHUNK)],
                             y_v.at[pl.ds(b * SH * CHUNK, SH * CHUNK)], sems[b])
            pltpu.async_copy(idx_hbm.at[pl.ds(c * CHUNK, CHUNK)],
                             idx_v.at[pl.ds(b * CHUNK, CHUNK)], sems[b])

        def chunk_wait(b):
            pltpu.make_async_copy(r_hbm.at[pl.ds(0, 3 * CHUNK)], r_v.at[pl.ds(b * 3 * CHUNK, 3 * CHUNK)], sems[b]).wait()
            pltpu.make_async_copy(y_hbm.at[pl.ds(0, SH * CHUNK)], y_v.at[pl.ds(b * SH * CHUNK, SH * CHUNK)], sems[b]).wait()
            pltpu.make_async_copy(idx_hbm.at[pl.ds(0, CHUNK)], idx_v.at[pl.ds(b * CHUNK, CHUNK)], sems[b]).wait()

        def process_chunk(b):
            @plsc.parallel_loop(0, NGRP, 1, unroll=4)
            def grp_body(j):
                pk = idx_v[pl.ds(b * CHUNK + j * 16, 16)]
                snd = pk & 0xFFFF
                rcv = lax.shift_right_logical(pk, 16)
                mf = plsc.load_gather(m_v, [snd])
                p = [mf * r_v[pl.ds((b * 3 + g) * CHUNK + j * 16, 16)] for g in range(3)]
                for k in range(SH):
                    msg = p[GOFK[k]] * y_v[pl.ds((b * SH + k) * CHUNK + j * 16, 16)]
                    plsc.addupdate_scatter(agg_v.at[pl.ds(k * N, N)], [rcv], msg)

        chunk_start(0, 0)
        chunk_start(1, 1)

        def pair_body(i, _):
            for b in range(2):
                c = 2 * i + b
                chunk_wait(b)
                process_chunk(b)

                @pl.when(c + 2 < NCHUNK)
                def _():
                    chunk_start(c + 2, b)
            return 0

        lax.fori_loop(0, NCHUNK // 2, pair_body, 0)

        pltpu.sync_copy(agg_v, out_hbm.at[pl.ds(f * SH * N, SH * N)])


_sc_agg = pl.kernel(
    _sc_agg_body,
    out_type=jax.ShapeDtypeStruct((F * SH * N,), jnp.float32),
    mesh=plsc.VectorSubcoreMesh(core_axis_name="c", subcore_axis_name="s"),
    compiler_params=pltpu.CompilerParams(needs_layout_passes=False),
    scratch_types=[
        pltpu.VMEM((N,), jnp.float32),
        pltpu.VMEM((2 * 3 * CHUNK,), jnp.float32),
        pltpu.VMEM((2 * SH * CHUNK,), jnp.float32),
        pltpu.VMEM((2 * CHUNK,), jnp.int32),
        pltpu.VMEM((SH * N,), jnp.float32),
        pltpu.SemaphoreType.DMA,
        pltpu.SemaphoreType.DMA,
    ],
)


def _head_kernel(nf1_0_ref, nf2_0_ref, Wro_ref, Wm1_ref, Wm2_ref, out_ref):
    out0 = nf1_0_ref[...] @ Wro_ref[...]
    h = nf2_0_ref[...] @ Wm1_ref[...]
    h = h * jax.nn.sigmoid(h)
    out1 = h @ Wm2_ref[...]
    out_ref[...] = jnp.stack([out0, out1], axis=1)


def kernel(positions, node_attrs, shifts, senders, receivers, W_embed, Wr1_0, Wr2_0, Wup_0, Wsc_0, Wprod_0, Wpattr_0, Wro_0, Wr1_1, Wr2_1, Wup_1, Wsc_1, Wprod_1, Wpattr_1, Wm1, Wm2):
    vec = positions[receivers] - positions[senders] + shifts
    r = jnp.sqrt(jnp.sum(vec ** 2, axis=-1) + 1e-18)
    u = vec / r[:, None]
    Y = _sph(u)
    ef = _bessel(r)
    h0 = node_attrs @ W_embed

    y_blocked = Y.T.reshape(SH, NCHUNK, CHUNK).transpose(1, 0, 2).reshape(-1)
    packed = jnp.bitwise_or(senders, receivers << 16)
    idx_blocked = packed.reshape(NCHUNK, CHUNK).reshape(-1)

    def interaction(h_scal, Wr1, Wr2, Wup, Wsc, Wprod, Wpattr):
        m_t = (h_scal @ Wup).T.reshape(-1)
        S = jax.nn.silu(ef @ Wr1)
        R_t = (S @ Wr2).T
        R_blk = R_t.reshape(3 * F, NCHUNK, CHUNK).transpose(1, 0, 2).reshape(-1)
        agg = _sc_agg(m_t, R_blk, y_blocked, idx_blocked)
        agg = agg.reshape(F, SH, N).transpose(2, 0, 1) / AVG
        sc = jnp.einsum('na,afk->nfk', node_attrs, Wsc)
        inv = jnp.mean(agg ** 2, axis=-1)
        wz = node_attrs @ Wpattr
        acc = jnp.zeros_like(agg)
        for nu in range(CORR):
            acc = acc + Wprod[nu][None, :, None] * agg * (inv[:, :, None] ** nu)
        return wz[:, :, None] * acc + sc

    nf1 = interaction(h0, Wr1_0, Wr2_0, Wup_0, Wsc_0, Wprod_0, Wpattr_0)
    nf2 = interaction(nf1[:, :, 0], Wr1_1, Wr2_1, Wup_1, Wsc_1, Wprod_1, Wpattr_1)

    out = pl.pallas_call(
        _head_kernel,
        out_shape=jax.ShapeDtypeStruct((N, 2, 1), jnp.float32),
    )(nf1[:, :, 0], nf2[:, :, 0], Wro_0, Wm1, Wm2)
    return out


# speedup vs baseline: 10.2042x; 1.3259x over previous
import jax
import jax.numpy as jnp
import numpy as np
from jax import lax
from jax.experimental import pallas as pl
from jax.experimental.pallas import tpu as pltpu
from jax.experimental.pallas import tpu_sc as plsc

N = 10000
E = 160000
A = 128
F = 64
NB = 8
RMAX = 5.0
CORR = 3
SH = 9
AVG = 16.0

CHUNK = 800
NCHUNK = E // CHUNK
NGRP = CHUNK // 16
assert CHUNK % 16 == 0 and E % CHUNK == 0 and NCHUNK % 2 == 0
GOFK = [0, 1, 1, 1, 2, 2, 2, 2, 2]


def _sph(u):
    x, y, z = u[..., 0], u[..., 1], u[..., 2]
    c1 = np.sqrt(3.0)
    c2 = np.sqrt(15.0)
    c3 = np.sqrt(5.0) / 2.0
    return jnp.stack([
        jnp.ones_like(x),
        c1 * x, c1 * y, c1 * z,
        c2 * x * y, c2 * y * z, c3 * (3.0 * z ** 2 - 1.0), c2 * x * z, (c2 / 2.0) * (x ** 2 - y ** 2),
    ], axis=-1)


def _bessel(r):
    n = jnp.arange(1, NB + 1, dtype=jnp.float32)
    r_ = r[..., None]
    b = jnp.sqrt(2.0 / RMAX) * jnp.sin(n * jnp.pi * r_ / RMAX) / (r_ + 1e-9)
    cut = 0.5 * (jnp.cos(jnp.pi * jnp.clip(r_ / RMAX, 0.0, 1.0)) + 1.0)
    return b * cut


def _sc_agg_body(m_hbm, r_hbm, y_hbm, idx_hbm, out_hbm,
                 m_v, r_v, y_v, idx_v, agg_v, sem0, sem1):
    cid = lax.axis_index("c")
    sid = lax.axis_index("s")
    wid = sid * 2 + cid
    sems = (sem0, sem1)

    for pass_i in range(2):
        f = pass_i * 32 + wid

        zero16 = jnp.zeros((16,), jnp.float32)

        @plsc.parallel_loop(0, (SH * N) // 16, 1, unroll=8)
        def zero_body(i):
            agg_v[pl.ds(i * 16, 16)] = zero16

        pltpu.sync_copy(m_hbm.at[pl.ds(f * N, N)], m_v)

        def chunk_start(c, b):
            for g in range(3):
                pltpu.async_copy(
                    r_hbm.at[pl.ds((f * 3 + g) * E + c * CHUNK, CHUNK)],
                    r_v.at[pl.ds((b * 3 + g) * CHUNK, CHUNK)], sems[b])
            pltpu.async_copy(y_hbm.at[pl.ds(c * SH * CHUNK, SH * CHUNK)],
                             y_v.at[pl.ds(b * SH * CHUNK, SH * CHUNK)], sems[b])
            pltpu.async_copy(idx_hbm.at[pl.ds(c * CHUNK, CHUNK)],
                             idx_v.at[pl.ds(b * CHUNK, CHUNK)], sems[b])

        def chunk_wait(b):
            for g in range(3):
                pltpu.make_async_copy(r_hbm.at[pl.ds(0, CHUNK)], r_v.at[pl.ds((b * 3 + g) * CHUNK, CHUNK)], sems[b]).wait()
            pltpu.make_async_copy(y_hbm.at[pl.ds(0, SH * CHUNK)], y_v.at[pl.ds(b * SH * CHUNK, SH * CHUNK)], sems[b]).wait()
            pltpu.make_async_copy(idx_hbm.at[pl.ds(0, CHUNK)], idx_v.at[pl.ds(b * CHUNK, CHUNK)], sems[b]).wait()

        def process_chunk(b):
            @plsc.parallel_loop(0, NGRP, 1, unroll=4)
            def grp_body(j):
                pk = idx_v[pl.ds(b * CHUNK + j * 16, 16)]
                snd = pk & 0xFFFF
                rcv = lax.shift_right_logical(pk, 16)
                mf = plsc.load_gather(m_v, [snd])
                p = [mf * r_v[pl.ds((b * 3 + g) * CHUNK + j * 16, 16)] for g in range(3)]
                for k in range(SH):
                    msg = p[GOFK[k]] * y_v[pl.ds((b * SH + k) * CHUNK + j * 16, 16)]
                    plsc.addupdate_scatter(agg_v.at[pl.ds(k * N, N)], [rcv], msg)

        chunk_start(0, 0)
        chunk_start(1, 1)

        def pair_body(i, _):
            for b in range(2):
                c = 2 * i + b
                chunk_wait(b)
                process_chunk(b)

                @pl.when(c + 2 < NCHUNK)
                def _():
                    chunk_start(c + 2, b)
            return 0

        lax.fori_loop(0, NCHUNK // 2, pair_body, 0)

        pltpu.sync_copy(agg_v, out_hbm.at[pl.ds(f * SH * N, SH * N)])


_sc_agg = pl.kernel(
    _sc_agg_body,
    out_type=jax.ShapeDtypeStruct((F * SH * N,), jnp.float32),
    mesh=plsc.VectorSubcoreMesh(core_axis_name="c", subcore_axis_name="s"),
    compiler_params=pltpu.CompilerParams(needs_layout_passes=False),
    scratch_types=[
        pltpu.VMEM((N,), jnp.float32),
        pltpu.VMEM((2 * 3 * CHUNK,), jnp.float32),
        pltpu.VMEM((2 * SH * CHUNK,), jnp.float32),
        pltpu.VMEM((2 * CHUNK,), jnp.int32),
        pltpu.VMEM((SH * N,), jnp.float32),
        pltpu.SemaphoreType.DMA,
        pltpu.SemaphoreType.DMA,
    ],
)


def _head_kernel(nf1_0_ref, nf2_0_ref, Wro_ref, Wm1_ref, Wm2_ref, out_ref):
    out0 = nf1_0_ref[...] @ Wro_ref[...]
    h = nf2_0_ref[...] @ Wm1_ref[...]
    h = h * jax.nn.sigmoid(h)
    out1 = h @ Wm2_ref[...]
    out_ref[...] = jnp.stack([out0, out1], axis=1)


def kernel(positions, node_attrs, shifts, senders, receivers, W_embed, Wr1_0, Wr2_0, Wup_0, Wsc_0, Wprod_0, Wpattr_0, Wro_0, Wr1_1, Wr2_1, Wup_1, Wsc_1, Wprod_1, Wpattr_1, Wm1, Wm2):
    vec = positions[receivers] - positions[senders] + shifts
    r = jnp.sqrt(jnp.sum(vec ** 2, axis=-1) + 1e-18)
    u = vec / r[:, None]
    Y = _sph(u)
    ef = _bessel(r)
    h0 = node_attrs @ W_embed

    y_blocked = Y.T.reshape(SH, NCHUNK, CHUNK).transpose(1, 0, 2).reshape(-1)
    packed = jnp.bitwise_or(senders, receivers << 16)
    idx_blocked = packed.reshape(NCHUNK, CHUNK).reshape(-1)

    def interaction(h_scal, Wr1, Wr2, Wup, Wsc, Wprod, Wpattr):
        m_t = (h_scal @ Wup).T.reshape(-1)
        S = jax.nn.silu(ef @ Wr1)
        R_t = (S @ Wr2).T.reshape(-1)
        agg = _sc_agg(m_t, R_t, y_blocked, idx_blocked)
        agg = agg.reshape(F, SH, N) / AVG
        inv = jnp.mean(agg ** 2, axis=1)
        agg0 = agg[:, 0, :]
        sc0 = (node_attrs @ Wsc[:, :, 0]).T
        wz = (node_attrs @ Wpattr).T
        acc = jnp.zeros_like(agg0)
        for nu in range(CORR):
            acc = acc + Wprod[nu][:, None] * agg0 * (inv ** nu)
        return (wz * acc + sc0).T

    nf1_0 = interaction(h0, Wr1_0, Wr2_0, Wup_0, Wsc_0, Wprod_0, Wpattr_0)
    nf2_0 = interaction(nf1_0, Wr1_1, Wr2_1, Wup_1, Wsc_1, Wprod_1, Wpattr_1)

    out = pl.pallas_call(
        _head_kernel,
        out_shape=jax.ShapeDtypeStruct((N, 2, 1), jnp.float32),
    )(nf1_0, nf2_0, Wro_0, Wm1, Wm2)
    return out


# speedup vs baseline: 11.3063x; 1.1080x over previous
import jax
import jax.numpy as jnp
import numpy as np
from jax import lax
from jax.experimental import pallas as pl
from jax.experimental.pallas import tpu as pltpu
from jax.experimental.pallas import tpu_sc as plsc

N = 10000
E = 160000
A = 128
F = 64
NB = 8
RMAX = 5.0
CORR = 3
SH = 9
AVG = 16.0

CHUNK = 800
NCHUNK = E // CHUNK
NGRP = CHUNK // 16
assert CHUNK % 16 == 0 and E % CHUNK == 0 and NCHUNK % 2 == 0
GOFK = [0, 1, 1, 1, 2, 2, 2, 2, 2]


def _sph(u):
    x, y, z = u[..., 0], u[..., 1], u[..., 2]
    c1 = np.sqrt(3.0)
    c2 = np.sqrt(15.0)
    c3 = np.sqrt(5.0) / 2.0
    return jnp.stack([
        jnp.ones_like(x),
        c1 * x, c1 * y, c1 * z,
        c2 * x * y, c2 * y * z, c3 * (3.0 * z ** 2 - 1.0), c2 * x * z, (c2 / 2.0) * (x ** 2 - y ** 2),
    ], axis=-1)


def _bessel(r):
    n = jnp.arange(1, NB + 1, dtype=jnp.float32)
    r_ = r[..., None]
    b = jnp.sqrt(2.0 / RMAX) * jnp.sin(n * jnp.pi * r_ / RMAX) / (r_ + 1e-9)
    cut = 0.5 * (jnp.cos(jnp.pi * jnp.clip(r_ / RMAX, 0.0, 1.0)) + 1.0)
    return b * cut


def _sc_agg_body(m_hbm, r_hbm, y_hbm, idx_hbm, out_hbm,
                 m_v, r_v, y_v, idx_v, agg_v, sem0, sem1):
    cid = lax.axis_index("c")
    sid = lax.axis_index("s")
    wid = sid * 2 + cid
    sems = (sem0, sem1)

    for pass_i in range(2):
        f = pass_i * 32 + wid

        zero16 = jnp.zeros((16,), jnp.float32)

        @plsc.parallel_loop(0, (SH * N) // 16, 1, unroll=8)
        def zero_body(i):
            agg_v[pl.ds(i * 16, 16)] = zero16

        pltpu.sync_copy(m_hbm.at[pl.ds(f * N, N)], m_v)

        def chunk_start(c, b):
            for g in range(3):
                pltpu.async_copy(
                    r_hbm.at[pl.ds((f * 3 + g) * E + c * CHUNK, CHUNK)],
                    r_v.at[pl.ds((b * 3 + g) * CHUNK, CHUNK)], sems[b])
            pltpu.async_copy(y_hbm.at[pl.ds(c * 3 * CHUNK, 3 * CHUNK)],
                             y_v.at[pl.ds(b * 3 * CHUNK, 3 * CHUNK)], sems[b])
            pltpu.async_copy(idx_hbm.at[pl.ds(c * CHUNK, CHUNK)],
                             idx_v.at[pl.ds(b * CHUNK, CHUNK)], sems[b])

        def chunk_wait(b):
            for g in range(3):
                pltpu.make_async_copy(r_hbm.at[pl.ds(0, CHUNK)], r_v.at[pl.ds((b * 3 + g) * CHUNK, CHUNK)], sems[b]).wait()
            pltpu.make_async_copy(y_hbm.at[pl.ds(0, 3 * CHUNK)], y_v.at[pl.ds(b * 3 * CHUNK, 3 * CHUNK)], sems[b]).wait()
            pltpu.make_async_copy(idx_hbm.at[pl.ds(0, CHUNK)], idx_v.at[pl.ds(b * CHUNK, CHUNK)], sems[b]).wait()

        c1 = np.float32(np.sqrt(3.0))
        c2 = np.float32(np.sqrt(15.0))
        c3 = np.float32(np.sqrt(5.0) / 2.0)

        def process_chunk(b):
            @plsc.parallel_loop(0, NGRP, 1, unroll=4)
            def grp_body(j):
                pk = idx_v[pl.ds(b * CHUNK + j * 16, 16)]
                snd = pk & 0xFFFF
                rcv = lax.shift_right_logical(pk, 16)
                mf = plsc.load_gather(m_v, [snd])
                p = [mf * r_v[pl.ds((b * 3 + g) * CHUNK + j * 16, 16)] for g in range(3)]
                ux = y_v[pl.ds((b * 3 + 0) * CHUNK + j * 16, 16)]
                uy = y_v[pl.ds((b * 3 + 1) * CHUNK + j * 16, 16)]
                uz = y_v[pl.ds((b * 3 + 2) * CHUNK + j * 16, 16)]
                p1c = p[1] * c1
                p2c = p[2] * c2
                msgs = (
                    p[0],
                    p1c * ux, p1c * uy, p1c * uz,
                    p2c * (ux * uy), p2c * (uy * uz),
                    (p[2] * c3) * (3.0 * (uz * uz) - 1.0),
                    p2c * (ux * uz),
                    (p[2] * (c2 * 0.5)) * (ux * ux - uy * uy),
                )
                for k in range(SH):
                    plsc.addupdate_scatter(agg_v.at[pl.ds(k * N, N)], [rcv], msgs[k])

        chunk_start(0, 0)
        chunk_start(1, 1)

        def pair_body(i, _):
            for b in range(2):
                c = 2 * i + b
                chunk_wait(b)
                process_chunk(b)

                @pl.when(c + 2 < NCHUNK)
                def _():
                    chunk_start(c + 2, b)
            return 0

        lax.fori_loop(0, NCHUNK // 2, pair_body, 0)

        pltpu.sync_copy(agg_v, out_hbm.at[pl.ds(f * SH * N, SH * N)])


_sc_agg = pl.kernel(
    _sc_agg_body,
    out_type=jax.ShapeDtypeStruct((F * SH * N,), jnp.float32),
    mesh=plsc.VectorSubcoreMesh(core_axis_name="c", subcore_axis_name="s"),
    compiler_params=pltpu.CompilerParams(needs_layout_passes=False),
    scratch_types=[
        pltpu.VMEM((N,), jnp.float32),
        pltpu.VMEM((2 * 3 * CHUNK,), jnp.float32),
        pltpu.VMEM((2 * 3 * CHUNK,), jnp.float32),
        pltpu.VMEM((2 * CHUNK,), jnp.int32),
        pltpu.VMEM((SH * N,), jnp.float32),
        pltpu.SemaphoreType.DMA,
        pltpu.SemaphoreType.DMA,
    ],
)


def _head_kernel(nf1_0_ref, nf2_0_ref, Wro_ref, Wm1_ref, Wm2_ref, out_ref):
    out0 = nf1_0_ref[...] @ Wro_ref[...]
    h = nf2_0_ref[...] @ Wm1_ref[...]
    h = h * jax.nn.sigmoid(h)
    out1 = h @ Wm2_ref[...]
    out_ref[...] = jnp.stack([out0, out1], axis=1)


def kernel(positions, node_attrs, shifts, senders, receivers, W_embed, Wr1_0, Wr2_0, Wup_0, Wsc_0, Wprod_0, Wpattr_0, Wro_0, Wr1_1, Wr2_1, Wup_1, Wsc_1, Wprod_1, Wpattr_1, Wm1, Wm2):
    vec = positions[receivers] - positions[senders] + shifts
    r = jnp.sqrt(jnp.sum(vec ** 2, axis=-1) + 1e-18)
    u = vec / r[:, None]
    ef = _bessel(r)
    h0 = node_attrs @ W_embed

    u_blocked = u.T.reshape(3, NCHUNK, CHUNK).transpose(1, 0, 2).reshape(-1)
    packed = jnp.bitwise_or(senders, receivers << 16)
    idx_blocked = packed.reshape(NCHUNK, CHUNK).reshape(-1)

    def interaction(h_scal, Wr1, Wr2, Wup, Wsc, Wprod, Wpattr):
        m_t = (h_scal @ Wup).T.reshape(-1)
        S = jax.nn.silu(ef @ Wr1)
        R_t = (S @ Wr2).T.reshape(-1)
        agg = _sc_agg(m_t, R_t, u_blocked, idx_blocked)
        agg = agg.reshape(F, SH, N) / AVG
        inv = jnp.mean(agg ** 2, axis=1)
        agg0 = agg[:, 0, :]
        sc0 = (node_attrs @ Wsc[:, :, 0]).T
        wz = (node_attrs @ Wpattr).T
        acc = jnp.zeros_like(agg0)
        for nu in range(CORR):
            acc = acc + Wprod[nu][:, None] * agg0 * (inv ** nu)
        return (wz * acc + sc0).T

    nf1_0 = interaction(h0, Wr1_0, Wr2_0, Wup_0, Wsc_0, Wprod_0, Wpattr_0)
    nf2_0 = interaction(nf1_0, Wr1_1, Wr2_1, Wup_1, Wsc_1, Wprod_1, Wpattr_1)

    out = pl.pallas_call(
        _head_kernel,
        out_shape=jax.ShapeDtypeStruct((N, 2, 1), jnp.float32),
    )(nf1_0, nf2_0, Wro_0, Wm1, Wm2)
    return out
